# Initial kernel scaffold; baseline (speedup 1.0000x reference)
import jax
import jax.numpy as jnp
from jax.experimental import pallas as pl


def kernel(input):
    raise NotImplementedError("write your pallas kernel here")


# speedup vs baseline: 7.9186x; 7.9186x over previous
import jax
import jax.numpy as jnp
from jax import lax
from jax.experimental import pallas as pl
from jax.experimental.pallas import tpu as pltpu

_NO = 8


def _body(x_ref, o_ref):
    x = x_ref[...]
    B, C = x.shape
    lane = lax.broadcasted_iota(jnp.int32, (B, C), 1)
    g = lane & (_NO - 1)

    def bfly(v, k):
        return jnp.where((g & k) == 0,
                         jnp.roll(v, -k, axis=1),
                         jnp.roll(v, k, axis=1))

    def groll(v, r):
        return jnp.where(g < _NO - r,
                         jnp.roll(v, -r, axis=1),
                         jnp.roll(v, _NO - r, axis=1))

    m = x
    for k in (1, 2, 4):
        m = jnp.maximum(m, bfly(m, k))
    score = jnp.where(x == m, g, _NO)
    for k in (1, 2, 4):
        score = jnp.minimum(score, bfly(score, k))
    y = x
    for s in (1, 2, 4):
        y = jnp.where((score & s) != 0, groll(y, s), y)
    o_ref[...] = y


def kernel(input):
    N, C = input.shape
    B = 256
    return pl.pallas_call(
        _body,
        grid=(N // B,),
        in_specs=[pl.BlockSpec((B, C), lambda i: (i, 0))],
        out_specs=pl.BlockSpec((B, C), lambda i: (i, 0)),
        out_shape=jax.ShapeDtypeStruct((N, C), input.dtype),
    )(input)


# speedup vs baseline: 9.1168x; 1.1513x over previous
import functools
import jax
import jax.numpy as jnp
from jax import lax
from jax.experimental import pallas as pl
from jax.experimental.pallas import tpu as pltpu
from jax.experimental.pallas import tpu_sc as plsc

_NO = 8
_L = 16
_SUB = _L * _NO


def _sc_align(x_flat, *, n_workers, chunk_elems, total_elems):
    per_worker = total_elems // n_workers
    n_chunks = per_worker // chunk_elems
    n_sub = chunk_elems // _SUB

    mesh = plsc.VectorSubcoreMesh(core_axis_name="c", subcore_axis_name="s")

    @functools.partial(
        pl.kernel,
        mesh=mesh,
        out_type=jax.ShapeDtypeStruct((total_elems,), jnp.float32),
        scratch_types=[
            pltpu.VMEM((chunk_elems,), jnp.float32),
            pltpu.VMEM((chunk_elems,), jnp.float32),
        ],
        compiler_params=pltpu.CompilerParams(needs_layout_passes=False),
    )
    def k(x_hbm, out_hbm, in_buf, out_buf):
        nc = lax.axis_size("c")
        wid = lax.axis_index("s") * nc + lax.axis_index("c")
        base = wid * per_worker

        iota = lax.iota(jnp.int32, _L)
        idx0 = [iota * _NO + o for o in range(_NO)]

        def compute_chunk(_):
            def sub_body(c, _):
                coff = c * _SUB
                idx = [idx0[o] + coff for o in range(_NO)]
                v = [plsc.load_gather(in_buf, [idx[o]]) for o in range(_NO)]
                m = v[0]
                d = jnp.zeros((_L,), jnp.int32)
                for o in range(1, _NO):
                    gt = v[o] > m
                    m = jnp.where(gt, v[o], m)
                    d = jnp.where(gt, jnp.full((_L,), o, jnp.int32), d)
                y = list(v)
                for b in (1, 2, 4):
                    take = (d & b) != 0
                    y = [jnp.where(take, y[(o + b) % _NO], y[o])
                         for o in range(_NO)]
                for o in range(_NO):
                    plsc.store_scatter(out_buf, [idx[o]], y[o])
                return _

            lax.fori_loop(0, n_sub, sub_body, None)

        def chunk_body(t, _):
            off = base + t * chunk_elems
            pltpu.sync_copy(x_hbm.at[pl.ds(off, chunk_elems)], in_buf)
            compute_chunk(None)
            pltpu.sync_copy(out_buf, out_hbm.at[pl.ds(off, chunk_elems)])
            return _

        lax.fori_loop(0, n_chunks, chunk_body, None)

    return k(x_flat)


def kernel(input):
    N, C = input.shape
    total = N * C
    out_flat = _sc_align(
        input.reshape(total),
        n_workers=32,
        chunk_elems=8 * 4096,
        total_elems=total,
    )
    return out_flat.reshape(N, C)


# speedup vs baseline: 10.2908x; 1.1288x over previous
import functools
import jax
import jax.numpy as jnp
from jax import lax
from jax.experimental import pallas as pl
from jax.experimental.pallas import tpu as pltpu
from jax.experimental.pallas import tpu_sc as plsc

_NO = 8
_L = 16
_SUB = _L * _NO


def _sc_align(x_flat, *, n_workers, chunk_elems, total_elems, unroll):
    per_worker = total_elems // n_workers
    n_chunks = per_worker // chunk_elems
    n_pairs = n_chunks // 2
    n_sub = chunk_elems // _SUB

    mesh = plsc.VectorSubcoreMesh(core_axis_name="c", subcore_axis_name="s")

    @functools.partial(
        pl.kernel,
        mesh=mesh,
        out_type=jax.ShapeDtypeStruct((total_elems,), jnp.float32),
        scratch_types=[
            pltpu.VMEM((chunk_elems,), jnp.float32),
            pltpu.VMEM((chunk_elems,), jnp.float32),
            pltpu.VMEM((chunk_elems,), jnp.float32),
            pltpu.VMEM((chunk_elems,), jnp.float32),
            pltpu.SemaphoreType.DMA,
            pltpu.SemaphoreType.DMA,
            pltpu.SemaphoreType.DMA,
            pltpu.SemaphoreType.DMA,
        ],
        compiler_params=pltpu.CompilerParams(needs_layout_passes=False),
    )
    def k(x_hbm, out_hbm, in0, in1, out0, out1, isem0, isem1, osem0, osem1):
        nc = lax.axis_size("c")
        wid = lax.axis_index("s") * nc + lax.axis_index("c")
        base = wid * per_worker

        iota = lax.iota(jnp.int32, _L)
        idx0 = [iota * _NO + o for o in range(_NO)]

        def start_in(t, buf, sem):
            pltpu.async_copy(x_hbm.at[pl.ds(base + t * chunk_elems,
                                            chunk_elems)], buf, sem)

        def wait_in(t, buf, sem):
            pltpu.make_async_copy(
                x_hbm.at[pl.ds(base + t * chunk_elems, chunk_elems)],
                buf, sem).wait()

        def start_out(t, buf, sem):
            pltpu.async_copy(buf, out_hbm.at[pl.ds(base + t * chunk_elems,
                                                   chunk_elems)], sem)

        def wait_out(t, buf, sem):
            pltpu.make_async_copy(
                buf, out_hbm.at[pl.ds(base + t * chunk_elems, chunk_elems)],
                sem).wait()

        def compute(in_buf, out_buf):
            @plsc.parallel_loop(0, n_sub, unroll=unroll)
            def _(c):
                coff = c * _SUB
                idx = [idx0[o] + coff for o in range(_NO)]
                v = [plsc.load_gather(in_buf, [idx[o]]) for o in range(_NO)]
                m = v[0]
                d = jnp.zeros((_L,), jnp.int32)
                for o in range(1, _NO):
                    gt = v[o] > m
                    m = jnp.where(gt, v[o], m)
                    d = jnp.where(gt, jnp.full((_L,), o, jnp.int32), d)
                y = list(v)
                for b in (1, 2, 4):
                    take = (d & b) != 0
                    y = [jnp.where(take, y[(o + b) % _NO], y[o])
                         for o in range(_NO)]
                for o in range(_NO):
                    plsc.store_scatter(out_buf, [idx[o]], y[o])

        def pair_body(p, carry):
            t0 = 2 * p
            t1 = t0 + 1
            start_in(t1, in1, isem1)
            wait_in(t0, in0, isem0)

            @pl.when(p > 0)
            def _():
                wait_out(t0 - 2, out0, osem0)

            compute(in0, out0)
            start_out(t0, out0, osem0)

            @pl.when(p + 1 < n_pairs)
            def _():
                start_in(t0 + 2, in0, isem0)

            wait_in(t1, in1, isem1)

            @pl.when(p > 0)
            def _():
                wait_out(t1 - 2, out1, osem1)

            compute(in1, out1)
            start_out(t1, out1, osem1)
            return carry

        start_in(0, in0, isem0)
        lax.fori_loop(0, n_pairs, pair_body, None)
        wait_out(n_chunks - 2, out0, osem0)
        wait_out(n_chunks - 1, out1, osem1)

    return k(x_flat)


def kernel(input):
    N, C = input.shape
    total = N * C
    out_flat = _sc_align(
        input.reshape(total),
        n_workers=32,
        chunk_elems=4 * 4096,
        total_elems=total,
        unroll=4,
    )
    return out_flat.reshape(N, C)


# speedup vs baseline: 34.7051x; 3.3724x over previous
import functools
import jax
import jax.numpy as jnp
from jax import lax
from jax.experimental import pallas as pl
from jax.experimental.pallas import tpu as pltpu
from jax.experimental.pallas import tpu_sc as plsc

_NO = 8
_L = 16
_SUB = _L * _NO


def _sc_align(x, *, n_workers, chunk_rows, unroll):
    n_rows, n_cols = x.shape
    rows_per_worker = n_rows // n_workers
    n_chunks = rows_per_worker // chunk_rows
    n_pairs = n_chunks // 2
    sub_per_row = n_cols // _SUB
    n_sub = chunk_rows * sub_per_row

    mesh = plsc.VectorSubcoreMesh(core_axis_name="c", subcore_axis_name="s")

    @functools.partial(
        pl.kernel,
        mesh=mesh,
        out_type=jax.ShapeDtypeStruct((n_rows, n_cols), jnp.float32),
        scratch_types=[
            pltpu.VMEM((chunk_rows, n_cols), jnp.float32),
            pltpu.VMEM((chunk_rows, n_cols), jnp.float32),
            pltpu.VMEM((chunk_rows, n_cols), jnp.float32),
            pltpu.VMEM((chunk_rows, n_cols), jnp.float32),
            pltpu.SemaphoreType.DMA,
            pltpu.SemaphoreType.DMA,
            pltpu.SemaphoreType.DMA,
            pltpu.SemaphoreType.DMA,
        ],
        compiler_params=pltpu.CompilerParams(needs_layout_passes=False),
    )
    def k(x_hbm, out_hbm, in0, in1, out0, out1, isem0, isem1, osem0, osem1):
        nc = lax.axis_size("c")
        wid = lax.axis_index("s") * nc + lax.axis_index("c")
        base = wid * rows_per_worker

        iota = lax.iota(jnp.int32, _L)
        col0 = [iota * _NO + o for o in range(_NO)]
        zero = jnp.zeros((_L,), jnp.int32)

        def start_in(t, buf, sem):
            pltpu.async_copy(
                x_hbm.at[pl.ds(base + t * chunk_rows, chunk_rows), :],
                buf, sem)

        def wait_in(t, buf, sem):
            pltpu.make_async_copy(
                x_hbm.at[pl.ds(base + t * chunk_rows, chunk_rows), :],
                buf, sem).wait()

        def start_out(t, buf, sem):
            pltpu.async_copy(
                buf, out_hbm.at[pl.ds(base + t * chunk_rows, chunk_rows), :],
                sem)

        def wait_out(t, buf, sem):
            pltpu.make_async_copy(
                buf, out_hbm.at[pl.ds(base + t * chunk_rows, chunk_rows), :],
                sem).wait()

        def compute(in_buf, out_buf):
            @plsc.parallel_loop(0, n_sub, unroll=unroll)
            def _(c):
                r = c // sub_per_row
                coff = (c % sub_per_row) * _SUB
                ridx = zero + r
                cidx = [col0[o] + coff for o in range(_NO)]
                v = [plsc.load_gather(in_buf, [ridx, cidx[o]])
                     for o in range(_NO)]
                m01 = jnp.maximum(v[0], v[1])
                m23 = jnp.maximum(v[2], v[3])
                m45 = jnp.maximum(v[4], v[5])
                m67 = jnp.maximum(v[6], v[7])
                m03 = jnp.maximum(m01, m23)
                m47 = jnp.maximum(m45, m67)
                m = jnp.maximum(m03, m47)
                take = m03 < m
                y = [jnp.where(take, v[(o + 4) % _NO], v[o])
                     for o in range(_NO)]
                take = jnp.maximum(y[0], y[1]) < m
                y = [jnp.where(take, y[(o + 2) % _NO], y[o])
                     for o in range(_NO)]
                take = y[0] < m
                y = [jnp.where(take, y[(o + 1) % _NO], y[o])
                     for o in range(_NO)]
                for o in range(_NO):
                    plsc.store_scatter(out_buf, [ridx, cidx[o]], y[o])

        def pair_body(p, carry):
            t0 = 2 * p
            t1 = t0 + 1
            start_in(t1, in1, isem1)
            wait_in(t0, in0, isem0)

            @pl.when(p > 0)
            def _():
                wait_out(t0 - 2, out0, osem0)

            compute(in0, out0)
            start_out(t0, out0, osem0)

            @pl.when(p + 1 < n_pairs)
            def _():
                start_in(t0 + 2, in0, isem0)

            wait_in(t1, in1, isem1)

            @pl.when(p > 0)
            def _():
                wait_out(t1 - 2, out1, osem1)

            compute(in1, out1)
            start_out(t1, out1, osem1)
            return carry

        start_in(0, in0, isem0)
        lax.fori_loop(0, n_pairs, pair_body, None)
        wait_out(n_chunks - 2, out0, osem0)
        wait_out(n_chunks - 1, out1, osem1)

    return k(x)


def kernel(input):
    return _sc_align(input, n_workers=32, chunk_rows=4, unroll=4)


# speedup vs baseline: 42.9044x; 1.2363x over previous
import functools
import jax
import jax.numpy as jnp
from jax import lax
from jax.experimental import pallas as pl
from jax.experimental.pallas import tpu as pltpu
from jax.experimental.pallas import tpu_sc as plsc

_NO = 8
_L = 16
_SUB = _L * _NO


def _sc_align(x, *, n_workers, chunk_rows, unroll):
    n_rows, n_cols = x.shape
    rows_per_worker = n_rows // n_workers
    n_chunks = rows_per_worker // chunk_rows
    n_pairs = n_chunks // 2
    sub_per_row = n_cols // _SUB
    n_sub = chunk_rows * sub_per_row

    mesh = plsc.VectorSubcoreMesh(core_axis_name="c", subcore_axis_name="s")

    @functools.partial(
        pl.kernel,
        mesh=mesh,
        out_type=jax.ShapeDtypeStruct((n_rows, n_cols), jnp.float32),
        scratch_types=[
            pltpu.VMEM((chunk_rows, n_cols), jnp.float32),
            pltpu.VMEM((chunk_rows, n_cols), jnp.float32),
            pltpu.VMEM((chunk_rows, n_cols), jnp.float32),
            pltpu.VMEM((chunk_rows, n_cols), jnp.float32),
            pltpu.SemaphoreType.DMA,
            pltpu.SemaphoreType.DMA,
            pltpu.SemaphoreType.DMA,
            pltpu.SemaphoreType.DMA,
        ],
        compiler_params=pltpu.CompilerParams(needs_layout_passes=False),
    )
    def k(x_hbm, out_hbm, in0, in1, out0, out1, isem0, isem1, osem0, osem1):
        nc = lax.axis_size("c")
        wid = lax.axis_index("s") * nc + lax.axis_index("c")
        base = wid * rows_per_worker

        iota = lax.iota(jnp.int32, _L)
        col0 = [iota * _NO + o for o in range(_NO)]
        zero = jnp.zeros((_L,), jnp.int32)

        def start_in(t, buf, sem):
            pltpu.async_copy(
                x_hbm.at[pl.ds(base + t * chunk_rows, chunk_rows), :],
                buf, sem)

        def wait_in(t, buf, sem):
            pltpu.make_async_copy(
                x_hbm.at[pl.ds(base + t * chunk_rows, chunk_rows), :],
                buf, sem).wait()

        def start_out(t, buf, sem):
            pltpu.async_copy(
                buf, out_hbm.at[pl.ds(base + t * chunk_rows, chunk_rows), :],
                sem)

        def wait_out(t, buf, sem):
            pltpu.make_async_copy(
                buf, out_hbm.at[pl.ds(base + t * chunk_rows, chunk_rows), :],
                sem).wait()

        def compute(in_buf, out_buf):
            @plsc.parallel_loop(0, n_sub, unroll=unroll)
            def _(c):
                r = c // sub_per_row
                coff = (c % sub_per_row) * _SUB
                src = in_buf.at[r, pl.ds(coff, _SUB)]
                dst = out_buf.at[r, pl.ds(coff, _SUB)]
                v = [plsc.load_gather(src, [col0[o]])
                     for o in range(_NO)]
                m01 = jnp.maximum(v[0], v[1])
                m23 = jnp.maximum(v[2], v[3])
                m45 = jnp.maximum(v[4], v[5])
                m67 = jnp.maximum(v[6], v[7])
                m03 = jnp.maximum(m01, m23)
                m47 = jnp.maximum(m45, m67)
                m = jnp.maximum(m03, m47)
                take = m03 < m
                y = [jnp.where(take, v[(o + 4) % _NO], v[o])
                     for o in range(_NO)]
                take = jnp.maximum(y[0], y[1]) < m
                y = [jnp.where(take, y[(o + 2) % _NO], y[o])
                     for o in range(_NO)]
                take = y[0] < m
                y = [jnp.where(take, y[(o + 1) % _NO], y[o])
                     for o in range(_NO)]
                for o in range(_NO):
                    plsc.store_scatter(dst, [col0[o]], y[o])

        def pair_body(p, carry):
            t0 = 2 * p
            t1 = t0 + 1
            start_in(t1, in1, isem1)
            wait_in(t0, in0, isem0)

            @pl.when(p > 0)
            def _():
                wait_out(t0 - 2, out0, osem0)

            compute(in0, out0)
            start_out(t0, out0, osem0)

            @pl.when(p + 1 < n_pairs)
            def _():
                start_in(t0 + 2, in0, isem0)

            wait_in(t1, in1, isem1)

            @pl.when(p > 0)
            def _():
                wait_out(t1 - 2, out1, osem1)

            compute(in1, out1)
            start_out(t1, out1, osem1)
            return carry

        start_in(0, in0, isem0)
        lax.fori_loop(0, n_pairs, pair_body, None)
        wait_out(n_chunks - 2, out0, osem0)
        wait_out(n_chunks - 1, out1, osem1)

    return k(x)


def kernel(input):
    return _sc_align(input, n_workers=32, chunk_rows=4, unroll=4)
